# Initial kernel scaffold; baseline (speedup 1.0000x reference)
import jax
import jax.numpy as jnp
from jax.experimental import pallas as pl


def kernel(features, memory_bank, influence_weight, distance_weight):
    raise NotImplementedError("write your pallas kernel here")


# speedup vs baseline: 4.8450x; 4.8450x over previous
import functools

import jax
import jax.numpy as jnp
from jax import lax
from jax.experimental import pallas as pl
from jax.experimental.pallas import tpu as pltpu

K = 9
NOISE_MIN = 0.01
NOISE_MAX = 0.5

QT = 512
MT = 1024
PAD = 128

_I32MAX = 2**31 - 1
_INF = float("inf")


def _k1_body(ff_ref, mbt_ref, topd_ref, topi_ref, w_ref, runv_ref, runi_ref,
             *, n_m):
    j = pl.program_id(1)

    @pl.when(j == 0)
    def _init():
        runv_ref[...] = jnp.full((QT, PAD), _INF, jnp.float32)
        runi_ref[...] = jnp.zeros((QT, PAD), jnp.int32)

    ff = ff_ref[...]
    mbt = mbt_ref[...]
    a2 = jnp.sum(ff * ff, axis=1, keepdims=True)
    b2 = jnp.sum(mbt * mbt, axis=0, keepdims=True)
    ab = jnp.dot(ff, mbt, preferred_element_type=jnp.float32)
    sq = a2 + b2 - 2.0 * ab
    colid = j * MT + lax.broadcasted_iota(jnp.int32, (QT, MT), 1)

    runv = runv_ref[...]
    runi = runi_ref[...]

    vs = []
    is_ = []
    for _ in range(K):
        m1 = jnp.min(sq, axis=1, keepdims=True)
        m2 = jnp.min(runv, axis=1, keepdims=True)
        m = jnp.minimum(m1, m2)
        eq1 = sq == m
        eq2 = runv == m
        ci = jnp.minimum(
            jnp.min(jnp.where(eq1, colid, _I32MAX), axis=1, keepdims=True),
            jnp.min(jnp.where(eq2, runi, _I32MAX), axis=1, keepdims=True),
        )
        sq = jnp.where(eq1 & (colid == ci), _INF, sq)
        runv = jnp.where(eq2 & (runi == ci), _INF, runv)
        vs.append(m)
        is_.append(ci)

    newv = jnp.concatenate(vs, axis=1)
    newi = jnp.concatenate(is_, axis=1)
    padv = jnp.full((QT, PAD - K), _INF, jnp.float32)
    padi = jnp.zeros((QT, PAD - K), jnp.int32)
    runv_ref[...] = jnp.concatenate([newv, padv], axis=1)
    runi_ref[...] = jnp.concatenate([newi, padi], axis=1)

    @pl.when(j == n_m - 1)
    def _finalize():
        d9 = jnp.sqrt(jnp.maximum(newv, 1e-12))
        w9 = 1.0 / d9
        zpad = jnp.zeros((QT, PAD - K), jnp.float32)
        topd_ref[...] = jnp.concatenate([d9, zpad], axis=1)
        w_ref[...] = jnp.concatenate([w9, zpad], axis=1)
        topi_ref[...] = jnp.concatenate([newi, padi], axis=1)


def _k1(ff, mbt, interpret=False):
    bn, d = ff.shape
    m = mbt.shape[1]
    n_q, n_m = bn // QT, m // MT
    return pl.pallas_call(
        functools.partial(_k1_body, n_m=n_m),
        grid=(n_q, n_m),
        in_specs=[
            pl.BlockSpec((QT, d), lambda i, j: (i, 0)),
            pl.BlockSpec((d, MT), lambda i, j: (0, j)),
        ],
        out_specs=[
            pl.BlockSpec((QT, PAD), lambda i, j: (i, 0)),
            pl.BlockSpec((QT, PAD), lambda i, j: (i, 0)),
            pl.BlockSpec((QT, PAD), lambda i, j: (i, 0)),
        ],
        out_shape=[
            jax.ShapeDtypeStruct((bn, PAD), jnp.float32),
            jax.ShapeDtypeStruct((bn, PAD), jnp.int32),
            jax.ShapeDtypeStruct((bn, PAD), jnp.float32),
        ],
        scratch_shapes=[
            pltpu.VMEM((QT, PAD), jnp.float32),
            pltpu.VMEM((QT, PAD), jnp.int32),
        ],
        compiler_params=pltpu.CompilerParams(
            dimension_semantics=("arbitrary", "arbitrary"),
        ),
        interpret=interpret,
    )(ff, mbt)


def _k3_body(ff_ref, v_ref, w_ref, topd_ref, iw_ref, dw_ref,
             infl_ref, noise_ref, *, bn, d):
    i = pl.program_id(0)
    ff = ff_ref[...]
    v = v_ref[...]
    w = w_ref[...]
    iw = iw_ref[...]
    dw = dw_ref[0, 0]

    s = jnp.sum(w, axis=1, keepdims=True)
    g = (ff * s - v) * (1.0 / K)
    infl = jnp.abs(g) * iw
    infl_ref[...] = infl

    mu = jnp.sum(infl, axis=1, keepdims=True) * (1.0 / d)
    diff = infl - mu
    var = jnp.sum(diff * diff, axis=1, keepdims=True) * (1.0 / (d - 1))
    inorm = diff / (jnp.sqrt(var) + 1e-8)

    topd = topd_ref[...]
    dsig_all = jnp.sum(topd, axis=1, keepdims=True) * (1.0 / K)
    dmu = jnp.sum(dsig_all) * (1.0 / bn)
    dvarnum = jnp.sum((dsig_all - dmu) ** 2)
    dstd = jnp.sqrt(dvarnum * d / (bn * d - 1))
    dsig_tile = jnp.sum(topd_ref[pl.ds(i * QT, QT), :], axis=1,
                        keepdims=True) * (1.0 / K)
    dnorm = (dsig_tile - dmu) / (dstd + 1e-8)

    comb = inorm + dw * dnorm
    noise_ref[...] = NOISE_MIN + (NOISE_MAX - NOISE_MIN) * jax.nn.sigmoid(comb)


def _k3(ff, v, w128, topd128, iw, dw, interpret=False):
    bn, d = ff.shape
    n_q = bn // QT
    return pl.pallas_call(
        functools.partial(_k3_body, bn=bn, d=d),
        grid=(n_q,),
        in_specs=[
            pl.BlockSpec((QT, d), lambda i: (i, 0)),
            pl.BlockSpec((QT, d), lambda i: (i, 0)),
            pl.BlockSpec((QT, PAD), lambda i: (i, 0)),
            pl.BlockSpec((bn, PAD), lambda i: (0, 0)),
            pl.BlockSpec((1, d), lambda i: (0, 0)),
            pl.BlockSpec(memory_space=pltpu.SMEM),
        ],
        out_specs=[
            pl.BlockSpec((QT, d), lambda i: (i, 0)),
            pl.BlockSpec((QT, d), lambda i: (i, 0)),
        ],
        out_shape=[
            jax.ShapeDtypeStruct((bn, d), jnp.float32),
            jax.ShapeDtypeStruct((bn, d), jnp.float32),
        ],
        compiler_params=pltpu.CompilerParams(
            dimension_semantics=("arbitrary",),
        ),
        interpret=interpret,
    )(ff, v, w128, topd128, iw, dw)


def _gather_v(memory_bank, idx9, w9):
    rows = memory_bank[idx9]
    bn9 = idx9.shape[0]
    return jnp.sum(rows.reshape(bn9 // K, K, -1) * w9.reshape(-1, K, 1), axis=1)


def _run(features, memory_bank, influence_weight, distance_weight,
         interpret=False):
    b, n, d = features.shape
    bn = b * n
    ff = features.reshape(bn, d)
    mbt = memory_bank.T

    topd128, topi128, w128 = _k1(ff, mbt, interpret=interpret)

    idx9 = topi128[:, :K].reshape(bn * K)
    w9 = w128[:, :K].reshape(bn * K)
    v = _gather_v(memory_bank, idx9, w9)

    iw = influence_weight.reshape(1, d)
    dw = distance_weight.reshape(1, 1)
    infl, noise = _k3(ff, v, w128, topd128, iw, dw, interpret=interpret)

    topk_d = topd128[:, :K].reshape(b, n, K)
    return (infl.reshape(b, n, d), noise.reshape(b, n, d), topk_d)


def kernel(features, memory_bank, influence_weight, distance_weight):
    return _run(features, memory_bank, influence_weight, distance_weight)


# speedup vs baseline: 5.5145x; 1.1382x over previous
import functools

import jax
import jax.numpy as jnp
from jax import lax
from jax.experimental import pallas as pl
from jax.experimental.pallas import tpu as pltpu
from jax.experimental.pallas import tpu_sc as plsc

K = 9
NOISE_MIN = 0.01
NOISE_MAX = 0.5

QT = 512
MT = 1024
PAD = 128

_I32MAX = 2**31 - 1
_INF = float("inf")


def _k1_body(ff_ref, mbt_ref, topd_ref, topi_ref, w_ref, runv_ref, runi_ref,
             *, n_m):
    j = pl.program_id(1)

    @pl.when(j == 0)
    def _init():
        runv_ref[...] = jnp.full((QT, PAD), _INF, jnp.float32)
        runi_ref[...] = jnp.zeros((QT, PAD), jnp.int32)

    ff = ff_ref[...]
    mbt = mbt_ref[...]
    a2 = jnp.sum(ff * ff, axis=1, keepdims=True)
    b2 = jnp.sum(mbt * mbt, axis=0, keepdims=True)
    ab = jnp.dot(ff, mbt, preferred_element_type=jnp.float32)
    sq = a2 + b2 - 2.0 * ab
    colid = j * MT + lax.broadcasted_iota(jnp.int32, (QT, MT), 1)

    runv = runv_ref[...]
    runi = runi_ref[...]

    vs = []
    is_ = []
    for _ in range(K):
        m1 = jnp.min(sq, axis=1, keepdims=True)
        m2 = jnp.min(runv, axis=1, keepdims=True)
        m = jnp.minimum(m1, m2)
        eq1 = sq == m
        eq2 = runv == m
        ci = jnp.minimum(
            jnp.min(jnp.where(eq1, colid, _I32MAX), axis=1, keepdims=True),
            jnp.min(jnp.where(eq2, runi, _I32MAX), axis=1, keepdims=True),
        )
        sq = jnp.where(eq1 & (colid == ci), _INF, sq)
        runv = jnp.where(eq2 & (runi == ci), _INF, runv)
        vs.append(m)
        is_.append(ci)

    newv = jnp.concatenate(vs, axis=1)
    newi = jnp.concatenate(is_, axis=1)
    padv = jnp.full((QT, PAD - K), _INF, jnp.float32)
    padi = jnp.zeros((QT, PAD - K), jnp.int32)
    runv_ref[...] = jnp.concatenate([newv, padv], axis=1)
    runi_ref[...] = jnp.concatenate([newi, padi], axis=1)

    @pl.when(j == n_m - 1)
    def _finalize():
        d9 = jnp.sqrt(jnp.maximum(newv, 1e-12))
        w9 = 1.0 / d9
        zpad = jnp.zeros((QT, PAD - K), jnp.float32)
        topd_ref[...] = jnp.concatenate([d9, zpad], axis=1)
        w_ref[...] = jnp.concatenate([w9, zpad], axis=1)
        topi_ref[...] = jnp.concatenate([newi, padi], axis=1)


def _k1(ff, mbt, interpret=False):
    bn, d = ff.shape
    m = mbt.shape[1]
    n_q, n_m = bn // QT, m // MT
    return pl.pallas_call(
        functools.partial(_k1_body, n_m=n_m),
        grid=(n_q, n_m),
        in_specs=[
            pl.BlockSpec((QT, d), lambda i, j: (i, 0)),
            pl.BlockSpec((d, MT), lambda i, j: (0, j)),
        ],
        out_specs=[
            pl.BlockSpec((QT, PAD), lambda i, j: (i, 0)),
            pl.BlockSpec((QT, PAD), lambda i, j: (i, 0)),
            pl.BlockSpec((QT, PAD), lambda i, j: (i, 0)),
        ],
        out_shape=[
            jax.ShapeDtypeStruct((bn, PAD), jnp.float32),
            jax.ShapeDtypeStruct((bn, PAD), jnp.int32),
            jax.ShapeDtypeStruct((bn, PAD), jnp.float32),
        ],
        scratch_shapes=[
            pltpu.VMEM((QT, PAD), jnp.float32),
            pltpu.VMEM((QT, PAD), jnp.int32),
        ],
        compiler_params=pltpu.CompilerParams(
            dimension_semantics=("arbitrary", "arbitrary"),
        ),
        interpret=interpret,
    )(ff, mbt)


def _k3_body(ff_ref, v_ref, w_ref, topd_ref, iw_ref, dw_ref,
             infl_ref, noise_ref, *, bn, d):
    i = pl.program_id(0)
    ff = ff_ref[...]
    v = v_ref[...]
    w = w_ref[...]
    iw = iw_ref[...]
    dw = dw_ref[0, 0]

    s = jnp.sum(w, axis=1, keepdims=True)
    g = (ff * s - v) * (1.0 / K)
    infl = jnp.abs(g) * iw
    infl_ref[...] = infl

    mu = jnp.sum(infl, axis=1, keepdims=True) * (1.0 / d)
    diff = infl - mu
    var = jnp.sum(diff * diff, axis=1, keepdims=True) * (1.0 / (d - 1))
    inorm = diff / (jnp.sqrt(var) + 1e-8)

    topd = topd_ref[...]
    dsig_all = jnp.sum(topd, axis=1, keepdims=True) * (1.0 / K)
    dmu = jnp.sum(dsig_all) * (1.0 / bn)
    dvarnum = jnp.sum((dsig_all - dmu) ** 2)
    dstd = jnp.sqrt(dvarnum * d / (bn * d - 1))
    dsig_tile = jnp.sum(topd_ref[pl.ds(i * QT, QT), :], axis=1,
                        keepdims=True) * (1.0 / K)
    dnorm = (dsig_tile - dmu) / (dstd + 1e-8)

    comb = inorm + dw * dnorm
    noise_ref[...] = NOISE_MIN + (NOISE_MAX - NOISE_MIN) * jax.nn.sigmoid(comb)


def _k3(ff, v, w128, topd128, iw, dw, interpret=False):
    bn, d = ff.shape
    n_q = bn // QT
    return pl.pallas_call(
        functools.partial(_k3_body, bn=bn, d=d),
        grid=(n_q,),
        in_specs=[
            pl.BlockSpec((QT, d), lambda i: (i, 0)),
            pl.BlockSpec((QT, d), lambda i: (i, 0)),
            pl.BlockSpec((QT, PAD), lambda i: (i, 0)),
            pl.BlockSpec((bn, PAD), lambda i: (0, 0)),
            pl.BlockSpec((1, d), lambda i: (0, 0)),
            pl.BlockSpec(memory_space=pltpu.SMEM),
        ],
        out_specs=[
            pl.BlockSpec((QT, d), lambda i: (i, 0)),
            pl.BlockSpec((QT, d), lambda i: (i, 0)),
        ],
        out_shape=[
            jax.ShapeDtypeStruct((bn, d), jnp.float32),
            jax.ShapeDtypeStruct((bn, d), jnp.float32),
        ],
        compiler_params=pltpu.CompilerParams(
            dimension_semantics=("arbitrary",),
        ),
        interpret=interpret,
    )(ff, v, w128, topd128, iw, dw)


def _gather_v(memory_bank, idx9, w9):
    bn9 = idx9.shape[0]
    bn = bn9 // K
    d = memory_bank.shape[1]
    info = plsc.get_sparse_core_info()
    nc, ns, nl = info.num_cores, info.num_subcores, info.num_lanes
    nw = nc * ns
    qpw = bn // nw
    c = 8
    nch = qpw // c
    mesh = plsc.VectorSubcoreMesh(core_axis_name="c", subcore_axis_name="s")

    @functools.partial(
        pl.kernel,
        mesh=mesh,
        out_type=jax.ShapeDtypeStruct((bn, d), jnp.float32),
        scratch_types=[
            pltpu.VMEM((c * K,), jnp.int32),
            pltpu.VMEM((c * K, 16), jnp.float32),
            pltpu.VMEM((c * K, d), jnp.float32),
            pltpu.VMEM((c, d), jnp.float32),
            pltpu.SemaphoreType.DMA,
        ],
    )
    def k2(mb_hbm, idx_hbm, w_hbm, out_hbm, idx_v, w_v, rows_v, acc_v, sem):
        wid = lax.axis_index("s") * nc + lax.axis_index("c")

        def chunk_body(t, carry):
            q0 = wid * qpw + t * c
            pltpu.sync_copy(idx_hbm.at[pl.ds(q0 * K, c * K)], idx_v)
            pltpu.sync_copy(w_hbm.at[pl.ds(q0 * K, c * K), :], w_v)
            pltpu.async_copy(mb_hbm.at[idx_v], rows_v, sem).wait()

            def q_body(q, carry2):
                wb = [w_v[q * K + r, :] for r in range(K)]

                def j_body(jj, carry3):
                    sl = pl.ds(jj * nl, nl)
                    acc = wb[0] * rows_v[q * K + 0, sl]
                    for r in range(1, K):
                        acc = acc + wb[r] * rows_v[q * K + r, sl]
                    acc_v[q, sl] = acc
                    return carry3

                lax.fori_loop(0, d // nl, j_body, 0)
                return carry2

            lax.fori_loop(0, c, q_body, 0)
            pltpu.sync_copy(acc_v, out_hbm.at[pl.ds(q0, c)])
            return carry

        lax.fori_loop(0, nch, chunk_body, 0)

    return k2(memory_bank, idx9, w9)


def _run(features, memory_bank, influence_weight, distance_weight,
         interpret=False):
    b, n, d = features.shape
    bn = b * n
    ff = features.reshape(bn, d)
    mbt = memory_bank.T

    topd128, topi128, w128 = _k1(ff, mbt, interpret=interpret)

    idx9 = topi128[:, :K].reshape(bn * K)
    w9 = w128[:, :K].reshape(bn * K)
    w9exp = jnp.broadcast_to(w9[:, None], (bn * K, 16))
    v = _gather_v(memory_bank, idx9, w9exp)

    iw = influence_weight.reshape(1, d)
    dw = distance_weight.reshape(1, 1)
    infl, noise = _k3(ff, v, w128, topd128, iw, dw, interpret=interpret)

    topk_d = topd128[:, :K].reshape(b, n, K)
    return (infl.reshape(b, n, d), noise.reshape(b, n, d), topk_d)


def kernel(features, memory_bank, influence_weight, distance_weight):
    return _run(features, memory_bank, influence_weight, distance_weight)
